# baseline (speedup 1.0000x reference)
import functools

import jax
import jax.numpy as jnp
from jax.experimental import pallas as pl
from jax.experimental.pallas import tpu as pltpu
from jax.experimental.pallas import tpu_sc as plsc

N, C, H, W = 16, 384, 32, 32
HW = H * W
CH = 192
K = HW // 4


def _score_body(x_ref, w1_ref, b1_ref, w2_ref, b2_ref, s_ref):
    X = x_ref[0]
    h = jnp.dot(w1_ref[...], X, preferred_element_type=jnp.float32)
    h = jnp.maximum(h + b1_ref[...], 0.0)
    s = jnp.dot(w2_ref[...], h, preferred_element_type=jnp.float32)
    s_ref[0] = s + b2_ref[...]


def _scores(fm3, W1, b1, W2, b2):
    return pl.pallas_call(
        _score_body,
        grid=(N,),
        in_specs=[
            pl.BlockSpec((1, C, HW), lambda n: (n, 0, 0)),
            pl.BlockSpec((CH, C), lambda n: (0, 0)),
            pl.BlockSpec((CH, 1), lambda n: (0, 0)),
            pl.BlockSpec((1, CH), lambda n: (0, 0)),
            pl.BlockSpec((1, 1), lambda n: (0, 0)),
        ],
        out_specs=pl.BlockSpec((1, 1, HW), lambda n: (n, 0, 0)),
        out_shape=jax.ShapeDtypeStruct((N, 1, HW), jnp.float32),
    )(fm3, W1, b1.reshape(CH, 1), W2, b2.reshape(1, 1))


def _icumsum(x):
    sh = 1
    while sh < HW:
        x = x + jnp.concatenate(
            [jnp.zeros((x.shape[0], sh), x.dtype), x[:, :-sh]], axis=1)
        sh *= 2
    return x


_HI = jax.lax.Precision.HIGHEST


def _topk_body(p_ref, idx_ref):
    p = p_ref[...]
    b = jax.lax.bitcast_convert_type(p, jnp.int32)
    key = jnp.where(b >= 0, b, jnp.int32(-2147483648) - b)

    lo = jnp.min(key, axis=1, keepdims=True)
    hi = jnp.max(key, axis=1, keepdims=True)

    def bis(_, lh):
        lo, hi = lh
        mid = lo + ((hi - lo + 1) >> 1)
        cnt = jnp.sum((key >= mid).astype(jnp.int32), axis=1, keepdims=True)
        ok = cnt >= K
        return jnp.where(ok, mid, lo), jnp.where(ok, hi, mid - 1)

    lo, hi = jax.lax.fori_loop(0, 31, bis, (lo, hi))
    v = lo
    gt = key > v
    eq = key == v
    ngt = jnp.sum(gt.astype(jnp.int32), axis=1, keepdims=True)
    eqc = _icumsum(eq.astype(jnp.int32))
    sel = gt | (eq & (eqc <= (K - ngt)))
    pos = _icumsum(sel.astype(jnp.int32)) - 1

    riota = jax.lax.broadcasted_iota(jnp.int32, (K, HW), 0)
    iiota = jax.lax.broadcasted_iota(jnp.int32, (1, HW), 1).astype(jnp.float32)
    eyeK = (jax.lax.broadcasted_iota(jnp.int32, (K, K), 0) ==
            jax.lax.broadcasted_iota(jnp.int32, (K, K), 1)).astype(jnp.float32)
    piota = jax.lax.broadcasted_iota(jnp.int32, (1, K), 1)
    dn_t = (((0,), (0,)), ((), ()))

    for bi in range(N):
        pos_b = pos[bi:bi + 1]
        sel_b = sel[bi:bi + 1]
        p_b = p[bi:bi + 1]
        M = ((jnp.broadcast_to(pos_b, (K, HW)) == riota)
             & jnp.broadcast_to(sel_b, (K, HW)))
        prob_c = jnp.sum(jnp.where(M, jnp.broadcast_to(p_b, (K, HW)), 0.0),
                         axis=1, keepdims=True)
        idx_c = jnp.sum(jnp.where(M, jnp.broadcast_to(iiota, (K, HW)), 0.0),
                        axis=1, keepdims=True)
        prob_r = jax.lax.dot_general(prob_c, eyeK, dn_t, precision=_HI)
        idx_r = jax.lax.dot_general(idx_c, eyeK, dn_t, precision=_HI)
        Cm = ((prob_r > prob_c)
              | ((prob_r == prob_c) & (idx_r < idx_c)))
        rank = jnp.sum(Cm.astype(jnp.int32), axis=1, keepdims=True)
        E = (jnp.broadcast_to(rank, (K, K)) == piota).astype(jnp.float32)
        out_r = jax.lax.dot_general(idx_c, E, dn_t, precision=_HI)
        idx_ref[bi:bi + 1, :] = out_r.astype(jnp.int32)


def _topk(probs):
    return pl.pallas_call(
        _topk_body,
        out_shape=jax.ShapeDtypeStruct((N, K), jnp.int32),
    )(probs)


_NW = 32
_RPW = (N * C) // _NW
_RB = 64
_NBLK = _RPW // _RB


def _gather_body(fm_ref, idx_ref, out_ref, idxv, rows, outv):
    cid = jax.lax.axis_index("c")
    sid = jax.lax.axis_index("s")
    wid = sid * 2 + cid
    n = wid // 2
    half = wid % 2
    base = n * C + half * (C // 2)
    pltpu.sync_copy(idx_ref.at[n], idxv)

    def block(bl, _):
        g0 = base + bl * _RB
        pltpu.sync_copy(fm_ref.at[pl.ds(g0 * HW, _RB * HW)], rows)

        def row(r, _):
            rbase = r * HW
            obase = r * K
            for t in range(K // 16):
                i16 = idxv[pl.ds(t * 16, 16)] + rbase
                outv[pl.ds(obase + t * 16, 16)] = plsc.load_gather(
                    rows, [i16])
            return 0

        jax.lax.fori_loop(0, _RB, row, 0)
        pltpu.sync_copy(outv, out_ref.at[pl.ds(g0 * K, _RB * K)])
        return 0

    jax.lax.fori_loop(0, _NBLK, block, 0)


@functools.partial(
    pl.kernel,
    mesh=plsc.VectorSubcoreMesh(core_axis_name="c", subcore_axis_name="s"),
    out_type=jax.ShapeDtypeStruct((N * C * K,), jnp.float32),
    scratch_types=[
        pltpu.VMEM((K,), jnp.int32),
        pltpu.VMEM((_RB * HW,), jnp.float32),
        pltpu.VMEM((_RB * K,), jnp.float32),
    ],
    compiler_params=pltpu.CompilerParams(needs_layout_passes=False),
)
def _sc_gather(fm_ref, idx_ref, out_ref, idxv, rows, outv):
    _gather_body(fm_ref, idx_ref, out_ref, idxv, rows, outv)


def kernel(feature_map, W1, b1, W2, b2):
    fm3 = feature_map.reshape(N, C, HW)
    scores = _scores(fm3, W1, b1, W2, b2)
    probs = jax.nn.softplus(scores)
    idx = _topk(probs.reshape(N, HW))
    out = _sc_gather(feature_map.reshape(N * C * HW), idx)
    return out.reshape(N, C, K, 1)


# speedup vs baseline: 1.0497x; 1.0497x over previous
import functools

import jax
import jax.numpy as jnp
from jax.experimental import pallas as pl
from jax.experimental.pallas import tpu as pltpu
from jax.experimental.pallas import tpu_sc as plsc

N, C, H, W = 16, 384, 32, 32
HW = H * W
CH = 192
K = HW // 4


def _score_body(x_ref, w1_ref, b1_ref, w2_ref, b2_ref, s_ref):
    X = x_ref[0]
    h = jnp.dot(w1_ref[...], X, preferred_element_type=jnp.float32)
    h = jnp.maximum(h + b1_ref[...], 0.0)
    s = jnp.dot(w2_ref[...], h, preferred_element_type=jnp.float32)
    s_ref[0] = s + b2_ref[...]


def _scores(fm3, W1, b1, W2, b2):
    return pl.pallas_call(
        _score_body,
        grid=(N,),
        in_specs=[
            pl.BlockSpec((1, C, HW), lambda n: (n, 0, 0)),
            pl.BlockSpec((CH, C), lambda n: (0, 0)),
            pl.BlockSpec((CH, 1), lambda n: (0, 0)),
            pl.BlockSpec((1, CH), lambda n: (0, 0)),
            pl.BlockSpec((1, 1), lambda n: (0, 0)),
        ],
        out_specs=pl.BlockSpec((1, 1, HW), lambda n: (n, 0, 0)),
        out_shape=jax.ShapeDtypeStruct((N, 1, HW), jnp.float32),
    )(fm3, W1, b1.reshape(CH, 1), W2, b2.reshape(1, 1))


def _icumsum(x):
    sh = 1
    while sh < HW:
        x = x + jnp.concatenate(
            [jnp.zeros((x.shape[0], sh), x.dtype), x[:, :-sh]], axis=1)
        sh *= 2
    return x


_HI = jax.lax.Precision.HIGHEST


def _topk_body(p_ref, idx_ref):
    p = p_ref[...]
    b = jax.lax.bitcast_convert_type(p, jnp.int32)
    key = jnp.where(b >= 0, b, jnp.int32(-2147483648) - b)

    lo = jnp.min(key, axis=1, keepdims=True)
    hi = jnp.max(key, axis=1, keepdims=True)

    def bis(_, lh):
        lo, hi = lh
        mid = lo + ((hi - lo + 1) >> 1)
        cnt = jnp.sum((key >= mid).astype(jnp.int32), axis=1, keepdims=True)
        ok = cnt >= K
        return jnp.where(ok, mid, lo), jnp.where(ok, hi, mid - 1)

    lo, hi = jax.lax.fori_loop(0, 31, bis, (lo, hi))
    v = lo
    gt = key > v
    eq = key == v
    ngt = jnp.sum(gt.astype(jnp.int32), axis=1, keepdims=True)
    eqc = _icumsum(eq.astype(jnp.int32))
    sel = gt | (eq & (eqc <= (K - ngt)))
    pos = _icumsum(sel.astype(jnp.int32)) - 1

    riota = jax.lax.broadcasted_iota(jnp.int32, (K, HW), 0)
    iiota = jax.lax.broadcasted_iota(jnp.int32, (1, HW), 1).astype(jnp.float32)
    eyeK = (jax.lax.broadcasted_iota(jnp.int32, (K, K), 0) ==
            jax.lax.broadcasted_iota(jnp.int32, (K, K), 1)).astype(jnp.float32)
    piota = jax.lax.broadcasted_iota(jnp.int32, (1, K), 1)
    dn_t = (((0,), (0,)), ((), ()))

    for bi in range(N):
        pos_b = pos[bi:bi + 1]
        sel_b = sel[bi:bi + 1]
        p_b = p[bi:bi + 1]
        M = ((jnp.broadcast_to(pos_b, (K, HW)) == riota)
             & jnp.broadcast_to(sel_b, (K, HW)))
        prob_c = jnp.sum(jnp.where(M, jnp.broadcast_to(p_b, (K, HW)), 0.0),
                         axis=1, keepdims=True)
        idx_c = jnp.sum(jnp.where(M, jnp.broadcast_to(iiota, (K, HW)), 0.0),
                        axis=1, keepdims=True)
        prob_r = jax.lax.dot_general(prob_c, eyeK, dn_t, precision=_HI)
        idx_r = jax.lax.dot_general(idx_c, eyeK, dn_t, precision=_HI)
        Cm = ((prob_r > prob_c)
              | ((prob_r == prob_c) & (idx_r < idx_c)))
        rank = jnp.sum(Cm.astype(jnp.int32), axis=1, keepdims=True)
        E = (jnp.broadcast_to(rank, (K, K)) == piota).astype(jnp.float32)
        out_r = jax.lax.dot_general(idx_c, E, dn_t, precision=_HI)
        oi = out_r.astype(jnp.int32)
        r0 = (2 * (bi // 8)) * 8 + bi % 8
        idx_ref[r0:r0 + 1, :] = oi[:, :128]
        idx_ref[r0 + 8:r0 + 9, :] = oi[:, 128:]


def _topk(probs):
    return pl.pallas_call(
        _topk_body,
        out_shape=jax.ShapeDtypeStruct((2 * N, 128), jnp.int32),
    )(probs)


_NW = 32
_RPW = (N * C) // _NW
_RB = 64
_NBLK = _RPW // _RB


def _gather_body(fm_ref, idx_ref, out_ref, idxv, rows, outv):
    cid = jax.lax.axis_index("c")
    sid = jax.lax.axis_index("s")
    wid = sid * 2 + cid
    n = wid // 2
    half = wid % 2
    base = n * C + half * (C // 2)
    pltpu.sync_copy(idx_ref, idxv)

    def block(bl, _):
        g0 = base + bl * _RB
        pltpu.sync_copy(fm_ref.at[pl.ds(g0, _RB), :], rows)

        def row(r, _):
            for t in range(K // 16):
                irow = (2 * (n // 8) + t // 8) * 8 + n % 8
                i16 = idxv[irow, pl.ds((t % 8) * 16, 16)]
                r16 = jnp.full((16,), r, jnp.int32)
                g16 = plsc.load_gather(rows, [r16, i16])
                outv[2 * r + (t // 8), pl.ds((t % 8) * 16, 16)] = g16
            return 0

        jax.lax.fori_loop(0, _RB, row, 0)
        pltpu.sync_copy(outv, out_ref.at[pl.ds(g0 * 2, _RB * 2), :])
        return 0

    jax.lax.fori_loop(0, _NBLK, block, 0)


@functools.partial(
    pl.kernel,
    mesh=plsc.VectorSubcoreMesh(core_axis_name="c", subcore_axis_name="s"),
    out_type=jax.ShapeDtypeStruct((2 * N * C, 128), jnp.float32),
    scratch_types=[
        pltpu.VMEM((2 * N, 128), jnp.int32),
        pltpu.VMEM((_RB, HW), jnp.float32),
        pltpu.VMEM((2 * _RB, 128), jnp.float32),
    ],
    compiler_params=pltpu.CompilerParams(
        needs_layout_passes=False, use_tc_tiling_on_sc=True),
)
def _sc_gather(fm_ref, idx_ref, out_ref, idxv, rows, outv):
    _gather_body(fm_ref, idx_ref, out_ref, idxv, rows, outv)


def kernel(feature_map, W1, b1, W2, b2):
    fm3 = feature_map.reshape(N, C, HW)
    scores = _scores(fm3, W1, b1, W2, b2)
    probs = jax.nn.softplus(scores)
    idxp = _topk(probs.reshape(N, HW))
    out = _sc_gather(feature_map.reshape(N * C, HW), idxp)
    return out.reshape(N, C, K, 1)


# speedup vs baseline: 1.0823x; 1.0310x over previous
import functools

import jax
import jax.numpy as jnp
from jax.experimental import pallas as pl
from jax.experimental.pallas import tpu as pltpu
from jax.experimental.pallas import tpu_sc as plsc

N, C, H, W = 16, 384, 32, 32
HW = H * W
CH = 192
K = HW // 4


def _score_body(x_ref, w1_ref, b1_ref, w2_ref, b2_ref, s_ref):
    X = x_ref[...]
    h = jnp.dot(w1_ref[...], X, preferred_element_type=jnp.float32)
    h = jnp.maximum(h + b1_ref[...], 0.0)
    s = jnp.dot(w2_ref[...], h, preferred_element_type=jnp.float32)
    s_ref[0] = s + b2_ref[...]


def _scores(fmr, W1, b1, W2, b2):
    return pl.pallas_call(
        _score_body,
        grid=(N,),
        in_specs=[
            pl.BlockSpec((C, HW), lambda n: (n, 0)),
            pl.BlockSpec((CH, C), lambda n: (0, 0)),
            pl.BlockSpec((CH, 1), lambda n: (0, 0)),
            pl.BlockSpec((1, CH), lambda n: (0, 0)),
            pl.BlockSpec((1, 1), lambda n: (0, 0)),
        ],
        out_specs=pl.BlockSpec((1, 1, HW), lambda n: (n, 0, 0)),
        out_shape=jax.ShapeDtypeStruct((N, 1, HW), jnp.float32),
    )(fmr, W1, b1.reshape(CH, 1), W2, b2.reshape(1, 1))


def _icumsum(x):
    sh = 1
    while sh < HW:
        x = x + jnp.concatenate(
            [jnp.zeros((x.shape[0], sh), x.dtype), x[:, :-sh]], axis=1)
        sh *= 2
    return x


_HI = jax.lax.Precision.HIGHEST


def _topk_body(p_ref, idx_ref):
    p = p_ref[...]
    b = jax.lax.bitcast_convert_type(p, jnp.int32)
    key = jnp.where(b >= 0, b, jnp.int32(-2147483648) - b)

    lo = jnp.min(key, axis=1, keepdims=True)
    hi = jnp.max(key, axis=1, keepdims=True)

    def bis(_, lh):
        lo, hi = lh
        mid = lo + ((hi - lo + 1) >> 1)
        cnt = jnp.sum((key >= mid).astype(jnp.int32), axis=1, keepdims=True)
        ok = cnt >= K
        return jnp.where(ok, mid, lo), jnp.where(ok, hi, mid - 1)

    lo, hi = jax.lax.fori_loop(0, 31, bis, (lo, hi))
    v = lo
    gt = key > v
    eq = key == v
    ngt = jnp.sum(gt.astype(jnp.int32), axis=1, keepdims=True)
    eqc = _icumsum(eq.astype(jnp.int32))
    sel = gt | (eq & (eqc <= (K - ngt)))
    pos = _icumsum(sel.astype(jnp.int32)) - 1

    riota = jax.lax.broadcasted_iota(jnp.int32, (K, HW), 0)
    iiota = jax.lax.broadcasted_iota(jnp.int32, (1, HW), 1).astype(jnp.float32)
    eyeK = (jax.lax.broadcasted_iota(jnp.int32, (K, K), 0) ==
            jax.lax.broadcasted_iota(jnp.int32, (K, K), 1)).astype(jnp.float32)
    piota = jax.lax.broadcasted_iota(jnp.int32, (1, K), 1)
    dn_t = (((0,), (0,)), ((), ()))

    for bi in range(N):
        pos_b = pos[bi:bi + 1]
        sel_b = sel[bi:bi + 1]
        p_b = p[bi:bi + 1]
        M = ((jnp.broadcast_to(pos_b, (K, HW)) == riota)
             & jnp.broadcast_to(sel_b, (K, HW)))
        prob_c = jnp.sum(jnp.where(M, jnp.broadcast_to(p_b, (K, HW)), 0.0),
                         axis=1, keepdims=True)
        idx_c = jnp.sum(jnp.where(M, jnp.broadcast_to(iiota, (K, HW)), 0.0),
                        axis=1, keepdims=True)
        prob_r = jax.lax.dot_general(prob_c, eyeK, dn_t, precision=_HI)
        idx_r = jax.lax.dot_general(idx_c, eyeK, dn_t, precision=_HI)
        Cm = ((prob_r > prob_c)
              | ((prob_r == prob_c) & (idx_r < idx_c)))
        rank = jnp.sum(Cm.astype(jnp.int32), axis=1, keepdims=True)
        E = (jnp.broadcast_to(rank, (K, K)) == piota).astype(jnp.float32)
        out_r = jax.lax.dot_general(idx_c, E, dn_t, precision=_HI)
        oi = out_r.astype(jnp.int32)
        r0 = (2 * (bi // 8)) * 8 + bi % 8
        idx_ref[r0:r0 + 1, :] = oi[:, :128]
        idx_ref[r0 + 8:r0 + 9, :] = oi[:, 128:]


def _topk(probs):
    return pl.pallas_call(
        _topk_body,
        out_shape=jax.ShapeDtypeStruct((2 * N, 128), jnp.int32),
    )(probs)


_NW = 32
_RPW = (N * C) // _NW
_RB = 64
_NBLK = _RPW // _RB


def _gather_body(fm_ref, idx_ref, out_ref, idxv, rows, outv):
    cid = jax.lax.axis_index("c")
    sid = jax.lax.axis_index("s")
    wid = sid * 2 + cid
    n = wid // 2
    half = wid % 2
    base = n * C + half * (C // 2)
    pltpu.sync_copy(idx_ref, idxv)

    def block(bl, _):
        g0 = base + bl * _RB
        pltpu.sync_copy(fm_ref.at[pl.ds(g0, _RB), :], rows)

        def row(r, _):
            for t in range(K // 16):
                irow = (2 * (n // 8) + t // 8) * 8 + n % 8
                i16 = idxv[irow, pl.ds((t % 8) * 16, 16)]
                r16 = jnp.full((16,), r, jnp.int32)
                g16 = plsc.load_gather(rows, [r16, i16])
                outv[r, pl.ds(t * 16, 16)] = g16
            return 0

        jax.lax.fori_loop(0, _RB, row, 0)
        c0 = half * (C // 2) + bl * _RB
        pltpu.sync_copy(outv, out_ref.at[n].at[pl.ds(c0, _RB), :])
        return 0

    jax.lax.fori_loop(0, _NBLK, block, 0)


@functools.partial(
    pl.kernel,
    mesh=plsc.VectorSubcoreMesh(core_axis_name="c", subcore_axis_name="s"),
    out_type=jax.ShapeDtypeStruct((N, C, K), jnp.float32),
    scratch_types=[
        pltpu.VMEM((2 * N, 128), jnp.int32),
        pltpu.VMEM((_RB, HW), jnp.float32),
        pltpu.VMEM((_RB, K), jnp.float32),
    ],
    compiler_params=pltpu.CompilerParams(
        needs_layout_passes=False, use_tc_tiling_on_sc=True),
)
def _sc_gather(fm_ref, idx_ref, out_ref, idxv, rows, outv):
    _gather_body(fm_ref, idx_ref, out_ref, idxv, rows, outv)


def kernel(feature_map, W1, b1, W2, b2):
    fmr = feature_map.reshape(N * C, HW)
    scores = _scores(fmr, W1, b1, W2, b2)
    probs = jax.nn.softplus(scores)
    idxp = _topk(probs.reshape(N, HW))
    out = _sc_gather(fmr, idxp)
    return out.reshape(N, C, K, 1)


# speedup vs baseline: 1.1255x; 1.0399x over previous
import functools

import jax
import jax.numpy as jnp
from jax.experimental import pallas as pl
from jax.experimental.pallas import tpu as pltpu
from jax.experimental.pallas import tpu_sc as plsc

N, C, H, W = 16, 384, 32, 32
HW = H * W
CH = 192
K = HW // 4


def _score_body(x_ref, w1_ref, b1_ref, w2_ref, b2_ref, s_ref):
    X = x_ref[...]
    h = jnp.dot(w1_ref[...], X, preferred_element_type=jnp.float32)
    h = jnp.maximum(h + b1_ref[...], 0.0)
    s = jnp.dot(w2_ref[...], h, preferred_element_type=jnp.float32)
    s_ref[0] = s + b2_ref[...]


def _scores(fmr, W1, b1, W2, b2):
    return pl.pallas_call(
        _score_body,
        grid=(N,),
        in_specs=[
            pl.BlockSpec((C, HW), lambda n: (n, 0)),
            pl.BlockSpec((CH, C), lambda n: (0, 0)),
            pl.BlockSpec((CH, 1), lambda n: (0, 0)),
            pl.BlockSpec((1, CH), lambda n: (0, 0)),
            pl.BlockSpec((1, 1), lambda n: (0, 0)),
        ],
        out_specs=pl.BlockSpec((1, 1, HW), lambda n: (n, 0, 0)),
        out_shape=jax.ShapeDtypeStruct((N, 1, HW), jnp.float32),
    )(fmr, W1, b1.reshape(CH, 1), W2, b2.reshape(1, 1))


def _icumsum(x):
    sh = 1
    while sh < HW:
        x = x + jnp.concatenate(
            [jnp.zeros((x.shape[0], sh), x.dtype), x[:, :-sh]], axis=1)
        sh *= 2
    return x


_HI = jax.lax.Precision.HIGHEST


def _topk_body(p_ref, idx_ref):
    p = p_ref[...]
    b = jax.lax.bitcast_convert_type(p, jnp.int32)
    key = jnp.where(b >= 0, b, jnp.int32(-2147483648) - b)

    lo = jnp.min(key, axis=1, keepdims=True)
    hi = jnp.max(key, axis=1, keepdims=True)

    def bis(_, lh):
        lo, hi = lh
        mid = lo + ((hi - lo + 1) >> 1)
        cnt = jnp.sum((key >= mid).astype(jnp.int32), axis=1, keepdims=True)
        ok = cnt >= K
        return jnp.where(ok, mid, lo), jnp.where(ok, hi, mid - 1)

    lo, hi = jax.lax.fori_loop(0, 31, bis, (lo, hi))
    v = lo
    gt = key > v
    eq = key == v
    ngt = jnp.sum(gt.astype(jnp.int32), axis=1, keepdims=True)
    eqc = _icumsum(eq.astype(jnp.int32))
    sel = gt | (eq & (eqc <= (K - ngt)))
    pos = _icumsum(sel.astype(jnp.int32)) - 1

    riota = jax.lax.broadcasted_iota(jnp.int32, (K, HW), 0)
    iiota = jax.lax.broadcasted_iota(jnp.int32, (1, HW), 1).astype(jnp.float32)
    eyeK = (jax.lax.broadcasted_iota(jnp.int32, (K, K), 0) ==
            jax.lax.broadcasted_iota(jnp.int32, (K, K), 1)).astype(jnp.float32)
    piota = jax.lax.broadcasted_iota(jnp.int32, (1, K), 1)
    dn_t = (((0,), (0,)), ((), ()))

    for bi in range(N):
        pos_b = pos[bi:bi + 1]
        sel_b = sel[bi:bi + 1]
        p_b = p[bi:bi + 1]
        M = ((jnp.broadcast_to(pos_b, (K, HW)) == riota)
             & jnp.broadcast_to(sel_b, (K, HW)))
        prob_c = jnp.sum(jnp.where(M, jnp.broadcast_to(p_b, (K, HW)), 0.0),
                         axis=1, keepdims=True)
        idx_c = jnp.sum(jnp.where(M, jnp.broadcast_to(iiota, (K, HW)), 0.0),
                        axis=1, keepdims=True)
        prob_r = jax.lax.dot_general(prob_c, eyeK, dn_t, precision=_HI)
        idx_r = jax.lax.dot_general(idx_c, eyeK, dn_t, precision=_HI)
        Cm = ((prob_r > prob_c)
              | ((prob_r == prob_c) & (idx_r < idx_c)))
        rank = jnp.sum(Cm.astype(jnp.int32), axis=1, keepdims=True)
        E = (jnp.broadcast_to(rank, (K, K)) == piota).astype(jnp.float32)
        out_r = jax.lax.dot_general(idx_c, E, dn_t, precision=_HI)
        oi = out_r.astype(jnp.int32)
        r0 = (2 * (bi // 8)) * 8 + bi % 8
        idx_ref[r0:r0 + 1, :] = oi[:, :128]
        idx_ref[r0 + 8:r0 + 9, :] = oi[:, 128:]


def _topk(probs):
    return pl.pallas_call(
        _topk_body,
        out_shape=jax.ShapeDtypeStruct((2 * N, 128), jnp.int32),
    )(probs)


_NW = 32
_RPW = (N * C) // _NW
_RB = 32
_NBLK = _RPW // _RB


def _gather_body(fm_ref, idx_ref, out_ref, idxv, idxc,
                 rows0, rows1, outv0, outv1, si0, si1, so0, so1):
    cid = jax.lax.axis_index("c")
    sid = jax.lax.axis_index("s")
    wid = sid * 2 + cid
    n = wid // 2
    half = wid % 2
    base = n * C + half * (C // 2)
    pltpu.sync_copy(idx_ref, idxv)
    for t in range(K // 16):
        irow = (2 * (n // 8) + t // 8) * 8 + n % 8
        idxc[pl.ds(t * 16, 16)] = idxv[irow, pl.ds((t % 8) * 16, 16)]

    rows_b = (rows0, rows1)
    out_b = (outv0, outv1)
    isem = (si0, si1)
    osem = (so0, so1)

    def in_copy(b):
        g0 = base + b * _RB
        return pltpu.make_async_copy(
            fm_ref.at[pl.ds(g0, _RB), :], rows_b[b % 2], isem[b % 2])

    def out_copy(b):
        c0 = half * (C // 2) + b * _RB
        return pltpu.make_async_copy(
            out_b[b % 2], out_ref.at[n].at[pl.ds(c0, _RB), :], osem[b % 2])

    in_copy(0).start()
    for b in range(_NBLK):
        if b + 1 < _NBLK:
            in_copy(b + 1).start()
        in_copy(b).wait()
        if b >= 2:
            out_copy(b - 2).wait()
        rows = rows_b[b % 2]
        outv = out_b[b % 2]

        def row(r, _, rows=rows, outv=outv):
            for t in range(K // 16):
                i16 = idxc[pl.ds(t * 16, 16)]
                r16 = jnp.full((16,), r, jnp.int32)
                outv[r, pl.ds(t * 16, 16)] = plsc.load_gather(
                    rows, [r16, i16])
            return 0

        jax.lax.fori_loop(0, _RB, row, 0)
        out_copy(b).start()
    out_copy(_NBLK - 2).wait()
    out_copy(_NBLK - 1).wait()


@functools.partial(
    pl.kernel,
    mesh=plsc.VectorSubcoreMesh(core_axis_name="c", subcore_axis_name="s"),
    out_type=jax.ShapeDtypeStruct((N, C, K), jnp.float32),
    scratch_types=[
        pltpu.VMEM((2 * N, 128), jnp.int32),
        pltpu.VMEM((K,), jnp.int32),
        pltpu.VMEM((_RB, HW), jnp.float32),
        pltpu.VMEM((_RB, HW), jnp.float32),
        pltpu.VMEM((_RB, K), jnp.float32),
        pltpu.VMEM((_RB, K), jnp.float32),
        pltpu.SemaphoreType.DMA,
        pltpu.SemaphoreType.DMA,
        pltpu.SemaphoreType.DMA,
        pltpu.SemaphoreType.DMA,
    ],
    compiler_params=pltpu.CompilerParams(
        needs_layout_passes=False, use_tc_tiling_on_sc=True),
)
def _sc_gather(fm_ref, idx_ref, out_ref, idxv, idxc,
               rows0, rows1, outv0, outv1, si0, si1, so0, so1):
    _gather_body(fm_ref, idx_ref, out_ref, idxv, idxc,
                 rows0, rows1, outv0, outv1, si0, si1, so0, so1)


def kernel(feature_map, W1, b1, W2, b2):
    fmr = feature_map.reshape(N * C, HW)
    scores = _scores(fmr, W1, b1, W2, b2)
    probs = jax.nn.softplus(scores)
    idxp = _topk(probs.reshape(N, HW))
    out = _sc_gather(fmr, idxp)
    return out.reshape(N, C, K, 1)
